# baseline (speedup 1.0000x reference)
import functools

import jax
import jax.numpy as jnp
from jax import lax
from jax.experimental import pallas as pl
from jax.experimental.pallas import tpu as pltpu
from jax.experimental.pallas import tpu_sc as plsc

D = 64
NW = 32
BPW = 128
NBUF = 2
NJ = BPW // 16


def kernel(x, W):
    B, S = x.shape
    T = W.reshape(500000, 128)
    xT = x.T

    mesh = plsc.VectorSubcoreMesh(core_axis_name="c", subcore_axis_name="s")

    @functools.partial(
        pl.kernel,
        out_type=jax.ShapeDtypeStruct((S, D // 8, B // BPW, 8, BPW),
                                      jnp.float32),
        mesh=mesh,
        compiler_params=pltpu.CompilerParams(
            use_tc_tiling_on_sc=False, needs_layout_passes=False
        ),
        scratch_types=[
            pltpu.VMEM((S, BPW), jnp.int32),
            pltpu.VMEM((NBUF, BPW), jnp.int32),
            pltpu.VMEM((NBUF, BPW), jnp.int32),
            pltpu.VMEM((NBUF, BPW, 128), jnp.float32),
            pltpu.VMEM((NBUF, D // 8, 8, BPW), jnp.float32),
            pltpu.SemaphoreType.DMA,
            pltpu.SemaphoreType.DMA,
            pltpu.SemaphoreType.DMA,
            pltpu.SemaphoreType.DMA,
        ],
    )
    def run(x_hbm, t_hbm, out_hbm, idx_v, kv_v, hb_v, pair_v, outs_v,
            g0, g1, o0, o1):
        wid = lax.axis_index("s") * 2 + lax.axis_index("c")
        b0 = wid * BPW
        pltpu.sync_copy(x_hbm.at[:, pl.ds(b0, BPW)], idx_v)
        gsems = (g0, g1)
        osems = (o0, o1)

        def compute_and_fire(s, b):
            for j in range(NJ):
                xv = idx_v[s, pl.ds(j * 16, 16)]
                kv_v[b, pl.ds(j * 16, 16)] = lax.shift_right_logical(xv, 1)
                hb_v[b, pl.ds(j * 16, 16)] = (xv & 1) * D
            pltpu.async_copy(t_hbm.at[kv_v.at[b]], pair_v.at[b], gsems[b])

        def process(b):
            pv = pair_v.at[b]
            for j in range(NJ):
                rows = lax.iota(jnp.int32, 16) + (j * 16)
                hb = hb_v[b, pl.ds(j * 16, 16)]
                for d in range(D):
                    v = plsc.load_gather(pv, [rows, hb + d])
                    outs_v[b, d // 8, d % 8, pl.ds(j * 16, 16)] = v

        compute_and_fire(0, 0)
        compute_and_fire(1, 1)

        def body(p, _):
            for b in range(NBUF):
                s = p * NBUF + b
                pltpu.make_async_copy(
                    t_hbm.at[pl.ds(0, BPW)], pair_v.at[b], gsems[b]
                ).wait()

                @pl.when(p != 0)
                def _():
                    pltpu.make_async_copy(
                        outs_v.at[b], out_hbm.at[0, :, wid], osems[b]
                    ).wait()

                process(b)
                pltpu.async_copy(
                    outs_v.at[b], out_hbm.at[s, :, wid], osems[b]
                )

                @pl.when(p != S // NBUF - 1)
                def _():
                    compute_and_fire(s + NBUF, b)
            return _

        lax.fori_loop(0, S // NBUF, body, None)
        for b in range(NBUF):
            pltpu.make_async_copy(
                outs_v.at[b], out_hbm.at[0, :, wid], osems[b]
            ).wait()

    r5 = run(xT, T)
    return r5.transpose(2, 4, 0, 1, 3).reshape(B, S, D)


# speedup vs baseline: 1.2540x; 1.2540x over previous
import functools

import jax
import jax.numpy as jnp
from jax import lax
from jax.experimental import pallas as pl
from jax.experimental.pallas import tpu as pltpu
from jax.experimental.pallas import tpu_sc as plsc

D = 64
NW = 32
BPW = 128
NBUF = 2
NJ = BPW // 16


def kernel(x, W):
    B, S = x.shape
    T = W.reshape(500000, 128)
    xT = x.T

    mesh = plsc.VectorSubcoreMesh(core_axis_name="c", subcore_axis_name="s")

    @functools.partial(
        pl.kernel,
        out_type=jax.ShapeDtypeStruct((S, D // 8, B // BPW, 8, BPW),
                                      jnp.float32),
        mesh=mesh,
        compiler_params=pltpu.CompilerParams(
            use_tc_tiling_on_sc=False, needs_layout_passes=False
        ),
        scratch_types=[
            pltpu.VMEM((S, BPW), jnp.int32),
            pltpu.VMEM((NBUF, BPW), jnp.int32),
            pltpu.VMEM((NBUF, BPW), jnp.int32),
            pltpu.VMEM((NBUF, BPW, 128), jnp.float32),
            pltpu.VMEM((NBUF, D // 8, 8, BPW), jnp.float32),
            pltpu.SemaphoreType.DMA,
            pltpu.SemaphoreType.DMA,
            pltpu.SemaphoreType.DMA,
            pltpu.SemaphoreType.DMA,
        ],
    )
    def run(x_hbm, t_hbm, out_hbm, idx_v, kv_v, hb_v, pair_v, outs_v,
            g0, g1, o0, o1):
        wid = lax.axis_index("s") * 2 + lax.axis_index("c")
        b0 = wid * BPW
        pltpu.sync_copy(x_hbm.at[:, pl.ds(b0, BPW)], idx_v)
        gsems = (g0, g1)
        osems = (o0, o1)

        def compute_and_fire(s, b):
            for j in range(NJ):
                xv = idx_v[s, pl.ds(j * 16, 16)]
                kv_v[b, pl.ds(j * 16, 16)] = lax.shift_right_logical(xv, 1)
                hb_v[b, pl.ds(j * 16, 16)] = (xv & 1) * D
            pltpu.async_copy(t_hbm.at[kv_v.at[b]], pair_v.at[b], gsems[b])

        def process(b):
            pv = pair_v.at[b]

            @plsc.parallel_loop(0, NJ, unroll=NJ)
            def _(j):
                rows = j * 16 + lax.iota(jnp.int32, 16)
                hb = hb_v[b, pl.ds(j * 16, 16)]
                for d in range(D):
                    v = plsc.load_gather(pv, [rows, hb + d])
                    outs_v[b, d // 8, d % 8, pl.ds(j * 16, 16)] = v

        compute_and_fire(0, 0)
        compute_and_fire(1, 1)

        def body(p, _):
            for b in range(NBUF):
                s = p * NBUF + b
                pltpu.make_async_copy(
                    t_hbm.at[pl.ds(0, BPW)], pair_v.at[b], gsems[b]
                ).wait()

                @pl.when(p != 0)
                def _():
                    pltpu.make_async_copy(
                        outs_v.at[b], out_hbm.at[0, :, wid], osems[b]
                    ).wait()

                process(b)
                pltpu.async_copy(
                    outs_v.at[b], out_hbm.at[s, :, wid], osems[b]
                )

                @pl.when(p != S // NBUF - 1)
                def _():
                    compute_and_fire(s + NBUF, b)
            return _

        lax.fori_loop(0, S // NBUF, body, None)
        for b in range(NBUF):
            pltpu.make_async_copy(
                outs_v.at[b], out_hbm.at[0, :, wid], osems[b]
            ).wait()

    r5 = run(xT, T)
    return r5.transpose(2, 4, 0, 1, 3).reshape(B, S, D)


# speedup vs baseline: 1.7786x; 1.4183x over previous
import functools

import jax
import jax.numpy as jnp
from jax import lax
from jax.experimental import pallas as pl
from jax.experimental.pallas import tpu as pltpu
from jax.experimental.pallas import tpu_sc as plsc

D = 64
NW = 32
BPW = 128
NBUF = 2
NJ = BPW // 16


def kernel(x, W):
    B, S = x.shape
    T = W.reshape(500000, 128)
    xT = x.T

    mesh = plsc.VectorSubcoreMesh(core_axis_name="c", subcore_axis_name="s")

    @functools.partial(
        pl.kernel,
        out_type=jax.ShapeDtypeStruct((S, D // 8, B // BPW, 8, BPW),
                                      jnp.float32),
        mesh=mesh,
        compiler_params=pltpu.CompilerParams(
            use_tc_tiling_on_sc=False, needs_layout_passes=False
        ),
        scratch_types=[
            pltpu.VMEM((S, BPW), jnp.int32),
            pltpu.VMEM((NBUF, BPW), jnp.int32),
            pltpu.VMEM((NBUF, BPW), jnp.int32),
            pltpu.VMEM((NBUF, BPW, 128), jnp.float32),
            pltpu.VMEM((NBUF, D // 8, 8, BPW + 1), jnp.float32),
            pltpu.SemaphoreType.DMA,
            pltpu.SemaphoreType.DMA,
            pltpu.SemaphoreType.DMA,
            pltpu.SemaphoreType.DMA,
        ],
    )
    def run(x_hbm, t_hbm, out_hbm, idx_v, kv_v, hb_v, pair_v, outs_v,
            g0, g1, o0, o1):
        wid = lax.axis_index("s") * 2 + lax.axis_index("c")
        b0 = wid * BPW
        pltpu.sync_copy(x_hbm.at[:, pl.ds(b0, BPW)], idx_v)
        gsems = (g0, g1)
        osems = (o0, o1)

        def compute_and_fire(s, b):
            for j in range(NJ):
                xv = idx_v[s, pl.ds(j * 16, 16)]
                kv_v[b, pl.ds(j * 16, 16)] = lax.shift_right_logical(xv, 1)
                hb_v[b, pl.ds(j * 16, 16)] = (xv & 1) * D
            pltpu.async_copy(t_hbm.at[kv_v.at[b]], pair_v.at[b], gsems[b])

        def process(b):
            pv = pair_v.at[b]
            iot = lax.iota(jnp.int32, 16)
            drv = iot & 7
            dbv = [lax.shift_right_logical(iot, 3) + 2 * k for k in range(4)]

            @plsc.parallel_loop(0, NJ, unroll=2)
            def _(jg):
                hv = hb_v[b, pl.ds(jg * 16, 16)]
                for i in range(16):
                    hs = hv[i]
                    colj = jnp.zeros((16,), jnp.int32) + (jg * 16 + i)
                    for k in range(D // 16):
                        v = pv[jg * 16 + i, pl.ds(hs + 16 * k, 16)]
                        plsc.store_scatter(
                            outs_v.at[b], [dbv[k], drv, colj], v
                        )

        compute_and_fire(0, 0)
        compute_and_fire(1, 1)

        def body(p, _):
            for b in range(NBUF):
                s = p * NBUF + b
                pltpu.make_async_copy(
                    t_hbm.at[pl.ds(0, BPW)], pair_v.at[b], gsems[b]
                ).wait()

                @pl.when(p != 0)
                def _():
                    pltpu.make_async_copy(
                        outs_v.at[b, :, :, pl.ds(0, BPW)],
                        out_hbm.at[0, :, wid], osems[b],
                    ).wait()

                process(b)
                pltpu.async_copy(
                    outs_v.at[b, :, :, pl.ds(0, BPW)],
                    out_hbm.at[s, :, wid], osems[b]
                )

                @pl.when(p != S // NBUF - 1)
                def _():
                    compute_and_fire(s + NBUF, b)
            return _

        lax.fori_loop(0, S // NBUF, body, None)
        for b in range(NBUF):
            pltpu.make_async_copy(
                outs_v.at[b, :, :, pl.ds(0, BPW)],
                out_hbm.at[0, :, wid], osems[b],
            ).wait()

    r5 = run(xT, T)
    return r5.transpose(2, 4, 0, 1, 3).reshape(B, S, D)


# speedup vs baseline: 2.2640x; 1.2729x over previous
import functools

import jax
import jax.numpy as jnp
from jax import lax
from jax.experimental import pallas as pl
from jax.experimental.pallas import tpu as pltpu
from jax.experimental.pallas import tpu_sc as plsc

D = 64
NW = 32
BPW = 128
NBUF = 2
NJ = BPW // 16


def kernel(x, W):
    B, S = x.shape
    T = jnp.pad(W, ((0, 0), (0, 128 - D)))
    xT = x.T

    mesh = plsc.VectorSubcoreMesh(core_axis_name="c", subcore_axis_name="s")

    @functools.partial(
        pl.kernel,
        out_type=jax.ShapeDtypeStruct((S, D // 8, B // BPW, 8, BPW),
                                      jnp.float32),
        mesh=mesh,
        compiler_params=pltpu.CompilerParams(
            use_tc_tiling_on_sc=False, needs_layout_passes=False
        ),
        scratch_types=[
            pltpu.VMEM((S, BPW), jnp.int32),
            pltpu.VMEM((NBUF, BPW, 128), jnp.float32),
            pltpu.VMEM((NBUF, D // 8, 8, BPW + 1), jnp.float32),
            pltpu.SemaphoreType.DMA,
            pltpu.SemaphoreType.DMA,
            pltpu.SemaphoreType.DMA,
            pltpu.SemaphoreType.DMA,
        ],
    )
    def run(x_hbm, t_hbm, out_hbm, idx_v, rows_v, outs_v, g0, g1, o0, o1):
        wid = lax.axis_index("s") * 2 + lax.axis_index("c")
        b0 = wid * BPW
        pltpu.sync_copy(x_hbm.at[:, pl.ds(b0, BPW)], idx_v)
        gsems = (g0, g1)
        osems = (o0, o1)

        def fire_gather(s, b):
            pltpu.async_copy(t_hbm.at[idx_v.at[s]], rows_v.at[b], gsems[b])

        def process(b):
            rv = rows_v.at[b]
            iot = lax.iota(jnp.int32, 16)
            drv = iot & 7
            dbv = [lax.shift_right_logical(iot, 3) + 2 * k
                   for k in range(D // 16)]

            @plsc.parallel_loop(0, NJ, unroll=2)
            def _(jg):
                for i in range(16):
                    j = jg * 16 + i
                    colj = jnp.zeros((16,), jnp.int32) + j
                    for k in range(D // 16):
                        v = rv[j, pl.ds(16 * k, 16)]
                        plsc.store_scatter(
                            outs_v.at[b], [dbv[k], drv, colj], v
                        )

        fire_gather(0, 0)
        fire_gather(1, 1)

        def body(p, _):
            for b in range(NBUF):
                s = p * NBUF + b
                pltpu.make_async_copy(
                    t_hbm.at[pl.ds(0, BPW)], rows_v.at[b], gsems[b]
                ).wait()

                @pl.when(p != 0)
                def _():
                    pltpu.make_async_copy(
                        outs_v.at[b, :, :, pl.ds(0, BPW)],
                        out_hbm.at[0, :, wid], osems[b],
                    ).wait()

                process(b)
                pltpu.async_copy(
                    outs_v.at[b, :, :, pl.ds(0, BPW)],
                    out_hbm.at[s, :, wid], osems[b]
                )

                @pl.when(p != S // NBUF - 1)
                def _():
                    fire_gather(s + NBUF, b)
            return _

        lax.fori_loop(0, S // NBUF, body, None)
        for b in range(NBUF):
            pltpu.make_async_copy(
                outs_v.at[b, :, :, pl.ds(0, BPW)],
                out_hbm.at[0, :, wid], osems[b],
            ).wait()

    r5 = run(xT, T)
    return r5.transpose(2, 4, 0, 1, 3).reshape(B, S, D)
